# baseline (speedup 1.0000x reference)
import functools

import jax
import jax.numpy as jnp
from jax import lax
from jax.experimental import pallas as pl
from jax.experimental.pallas import tpu as pltpu
from jax.experimental.pallas import tpu_sc as plsc

N_NODES = 16384
OBS_DIM = 4096
NUM_EXPERTS = 10
TOP_K = 3

NUM_WORKERS = 32
N_TC = 0
N_SC = N_NODES - N_TC
ROWS_PER_TEC = N_SC // NUM_WORKERS
CHUNK = 8
N_CHUNKS = ROWS_PER_TEC // CHUNK
N_STRIPS = OBS_DIM // 16

ROW_BLOCK = 1024


def _sc_reduce_body(x_hbm, out_hbm, buf0, buf1, acc, sem0, sem1):
    c = lax.axis_index("c")
    s = lax.axis_index("s")
    wid = s * 2 + c
    base = N_TC + wid * ROWS_PER_TEC

    zeros16 = jnp.zeros((16,), jnp.float32)

    def _zero(i, carry):
        acc[pl.ds(i * 16, 16)] = zeros16
        return carry

    lax.fori_loop(0, N_STRIPS, _zero, 0)

    def _start(g, buf, sem):
        pltpu.make_async_copy(
            x_hbm.at[pl.ds(base + g * CHUNK, CHUNK)], buf, sem).start()

    def _accumulate(buf):
        def _strip(i, carry):
            off = pl.ds(i * 16, 16)
            a = acc[off]
            for r in range(CHUNK):
                a = a + buf[r, off]
            acc[off] = a
            return carry
        lax.fori_loop(0, N_STRIPS, _strip, 0)

    _start(0, buf0, sem0)
    _start(1, buf1, sem1)

    def _pair(gp, carry):
        g0 = gp * 2

        pltpu.make_async_copy(
            x_hbm.at[pl.ds(base + g0 * CHUNK, CHUNK)], buf0, sem0).wait()
        _accumulate(buf0)

        @pl.when(g0 + 2 < N_CHUNKS)
        def _():
            _start(g0 + 2, buf0, sem0)

        pltpu.make_async_copy(
            x_hbm.at[pl.ds(base + (g0 + 1) * CHUNK, CHUNK)], buf1, sem1).wait()
        _accumulate(buf1)

        @pl.when(g0 + 3 < N_CHUNKS)
        def _():
            _start(g0 + 3, buf1, sem1)

        return carry

    lax.fori_loop(0, N_CHUNKS // 2, _pair, 0)

    pltpu.sync_copy(acc, out_hbm.at[wid])


@functools.partial(
    pl.kernel,
    mesh=plsc.VectorSubcoreMesh(core_axis_name="c", subcore_axis_name="s"),
    out_type=jax.ShapeDtypeStruct((NUM_WORKERS, OBS_DIM), jnp.float32),
    scratch_types=[
        pltpu.VMEM((CHUNK, OBS_DIM), jnp.float32),
        pltpu.VMEM((CHUNK, OBS_DIM), jnp.float32),
        pltpu.VMEM((OBS_DIM,), jnp.float32),
        pltpu.SemaphoreType.DMA,
        pltpu.SemaphoreType.DMA,
    ],
)
def _sc_reduce(x_hbm, out_hbm, buf0, buf1, acc, sem0, sem1):
    _sc_reduce_body(x_hbm, out_hbm, buf0, buf1, acc, sem0, sem1)


def _head_kernel(sc_ref, w_ref, b_ref, eps_ref, w_out, l_out, v_out, i_out):
    total = jnp.sum(sc_ref[...], axis=0, keepdims=True)
    gs = total * (1.0 / N_NODES)
    both = jnp.dot(gs, w_ref[...],
                   preferred_element_type=jnp.float32) + b_ref[...]
    clean = both[:, :NUM_EXPERTS]
    raw = both[:, NUM_EXPERTS:]
    noise_std = jnp.logaddexp(raw, 0.0)
    logits = clean + eps_ref[...] * noise_std
    m = jnp.max(logits, axis=-1, keepdims=True)
    e = jnp.exp(logits - m)
    weights = e / jnp.sum(e, axis=-1, keepdims=True)
    l_out[...] = logits
    w_out[...] = weights

    idxs = lax.broadcasted_iota(jnp.int32, (1, NUM_EXPERTS), 1)
    cur = weights
    vals, inds = [], []
    for _ in range(TOP_K):
        v = jnp.max(cur, axis=-1, keepdims=True)
        a = jnp.min(jnp.where(cur == v, idxs, NUM_EXPERTS),
                    axis=-1, keepdims=True)
        cur = jnp.where(idxs == a, -jnp.inf, cur)
        vals.append(v)
        inds.append(a)
    v_out[...] = jnp.concatenate(vals, axis=-1)
    i_out[...] = jnp.concatenate(inds, axis=-1)


def _head(partials, w_both, b_both, eps):
    n_part = partials.shape[0]
    return pl.pallas_call(
        _head_kernel,
        out_shape=[
            jax.ShapeDtypeStruct((1, NUM_EXPERTS), jnp.float32),
            jax.ShapeDtypeStruct((1, NUM_EXPERTS), jnp.float32),
            jax.ShapeDtypeStruct((1, TOP_K), jnp.float32),
            jax.ShapeDtypeStruct((1, TOP_K), jnp.int32),
        ],
    )(partials, w_both, b_both, eps)


@jax.jit
def _run(ip_x, w_both, b_both, eps):
    sc_part = _sc_reduce(ip_x)
    weights, logits, top_k_vals, top_k_indices = _head(
        sc_part, w_both, b_both, eps)
    return weights, logits, top_k_vals, top_k_indices


def kernel(ip_x, w_gating, b_gating, w_noise, b_noise, top_k):
    eps = jax.random.normal(jax.random.key(42), (1, NUM_EXPERTS),
                            dtype=jnp.float32)
    w_both = jnp.concatenate([w_gating, w_noise], axis=1)
    b_both = jnp.concatenate([b_gating, b_noise]).reshape(1, 2 * NUM_EXPERTS)
    del top_k
    return _run(ip_x, w_both, b_both, eps)


# speedup vs baseline: 2.2039x; 2.2039x over previous
import functools

import jax
import jax.numpy as jnp
from jax import lax
from jax.experimental import pallas as pl
from jax.experimental.pallas import tpu as pltpu
from jax.experimental.pallas import tpu_sc as plsc

N_NODES = 16384
OBS_DIM = 4096
NUM_EXPERTS = 10
TOP_K = 3

NUM_WORKERS = 32
N_TC = 12288
N_SC = N_NODES - N_TC
ROWS_PER_TEC = N_SC // NUM_WORKERS
CHUNK = 8
N_CHUNKS = ROWS_PER_TEC // CHUNK
N_STRIPS = OBS_DIM // 16

ROW_BLOCK = 1024


def _sc_reduce_body(x_hbm, out_hbm, buf0, buf1, acc, sem0, sem1):
    c = lax.axis_index("c")
    s = lax.axis_index("s")
    wid = s * 2 + c
    base = N_TC + wid * ROWS_PER_TEC

    zeros16 = jnp.zeros((16,), jnp.float32)

    def _zero(i, carry):
        acc[pl.ds(i * 16, 16)] = zeros16
        return carry

    lax.fori_loop(0, N_STRIPS, _zero, 0)

    def _start(g, buf, sem):
        pltpu.make_async_copy(
            x_hbm.at[pl.ds(base + g * CHUNK, CHUNK)], buf, sem).start()

    def _accumulate(buf):
        def _strip(i, carry):
            off = pl.ds(i * 16, 16)
            a = acc[off]
            for r in range(CHUNK):
                a = a + buf[r, off]
            acc[off] = a
            return carry
        lax.fori_loop(0, N_STRIPS, _strip, 0)

    _start(0, buf0, sem0)
    _start(1, buf1, sem1)

    def _pair(gp, carry):
        g0 = gp * 2

        pltpu.make_async_copy(
            x_hbm.at[pl.ds(base + g0 * CHUNK, CHUNK)], buf0, sem0).wait()
        _accumulate(buf0)

        @pl.when(g0 + 2 < N_CHUNKS)
        def _():
            _start(g0 + 2, buf0, sem0)

        pltpu.make_async_copy(
            x_hbm.at[pl.ds(base + (g0 + 1) * CHUNK, CHUNK)], buf1, sem1).wait()
        _accumulate(buf1)

        @pl.when(g0 + 3 < N_CHUNKS)
        def _():
            _start(g0 + 3, buf1, sem1)

        return carry

    lax.fori_loop(0, N_CHUNKS // 2, _pair, 0)

    pltpu.sync_copy(acc, out_hbm.at[wid])


@functools.partial(
    pl.kernel,
    mesh=plsc.VectorSubcoreMesh(core_axis_name="c", subcore_axis_name="s"),
    out_type=jax.ShapeDtypeStruct((NUM_WORKERS, OBS_DIM), jnp.float32),
    scratch_types=[
        pltpu.VMEM((CHUNK, OBS_DIM), jnp.float32),
        pltpu.VMEM((CHUNK, OBS_DIM), jnp.float32),
        pltpu.VMEM((OBS_DIM,), jnp.float32),
        pltpu.SemaphoreType.DMA,
        pltpu.SemaphoreType.DMA,
    ],
)
def _sc_reduce(x_hbm, out_hbm, buf0, buf1, acc, sem0, sem1):
    _sc_reduce_body(x_hbm, out_hbm, buf0, buf1, acc, sem0, sem1)


def _tc_reduce_kernel(x_ref, out_ref):
    step = pl.program_id(0)

    @pl.when(step == 0)
    def _init():
        out_ref[...] = jnp.zeros_like(out_ref)

    out_ref[...] += jnp.sum(x_ref[...], axis=0, keepdims=True)


def _tc_reduce(ip_x):
    return pl.pallas_call(
        _tc_reduce_kernel,
        grid=(N_TC // ROW_BLOCK,),
        in_specs=[pl.BlockSpec((ROW_BLOCK, OBS_DIM), lambda i: (i, 0))],
        out_specs=pl.BlockSpec((1, OBS_DIM), lambda i: (0, 0)),
        out_shape=jax.ShapeDtypeStruct((1, OBS_DIM), jnp.float32),
        compiler_params=pltpu.CompilerParams(
            dimension_semantics=("arbitrary",),
        ),
    )(ip_x)


def _head_kernel(sc_ref, tc_ref, w_ref, b_ref, eps_ref,
                 w_out, l_out, v_out, i_out):
    total = (jnp.sum(sc_ref[...], axis=0, keepdims=True)
             + tc_ref[...])
    gs = total * (1.0 / N_NODES)
    both = jnp.dot(gs, w_ref[...],
                   preferred_element_type=jnp.float32) + b_ref[...]
    clean = both[:, :NUM_EXPERTS]
    raw = both[:, NUM_EXPERTS:]
    noise_std = jnp.logaddexp(raw, 0.0)
    logits = clean + eps_ref[...] * noise_std
    m = jnp.max(logits, axis=-1, keepdims=True)
    e = jnp.exp(logits - m)
    weights = e / jnp.sum(e, axis=-1, keepdims=True)
    l_out[...] = logits
    w_out[...] = weights

    idxs = lax.broadcasted_iota(jnp.int32, (1, NUM_EXPERTS), 1)
    cur = weights
    vals, inds = [], []
    for _ in range(TOP_K):
        v = jnp.max(cur, axis=-1, keepdims=True)
        a = jnp.min(jnp.where(cur == v, idxs, NUM_EXPERTS),
                    axis=-1, keepdims=True)
        cur = jnp.where(idxs == a, -jnp.inf, cur)
        vals.append(v)
        inds.append(a)
    v_out[...] = jnp.concatenate(vals, axis=-1)
    i_out[...] = jnp.concatenate(inds, axis=-1)


def _head(sc_part, tc_part, w_both, b_both, eps):
    return pl.pallas_call(
        _head_kernel,
        out_shape=[
            jax.ShapeDtypeStruct((1, NUM_EXPERTS), jnp.float32),
            jax.ShapeDtypeStruct((1, NUM_EXPERTS), jnp.float32),
            jax.ShapeDtypeStruct((1, TOP_K), jnp.float32),
            jax.ShapeDtypeStruct((1, TOP_K), jnp.int32),
        ],
    )(sc_part, tc_part, w_both, b_both, eps)


@jax.jit
def _run(ip_x, w_both, b_both, eps):
    sc_part = _sc_reduce(ip_x)
    tc_part = _tc_reduce(ip_x)
    weights, logits, top_k_vals, top_k_indices = _head(
        sc_part, tc_part, w_both, b_both, eps)
    return weights, logits, top_k_vals, top_k_indices


def kernel(ip_x, w_gating, b_gating, w_noise, b_noise, top_k):
    eps = jax.random.normal(jax.random.key(42), (1, NUM_EXPERTS),
                            dtype=jnp.float32)
    w_both = jnp.concatenate([w_gating, w_noise], axis=1)
    b_both = jnp.concatenate([b_gating, b_noise]).reshape(1, 2 * NUM_EXPERTS)
    del top_k
    return _run(ip_x, w_both, b_both, eps)


# speedup vs baseline: 2.6501x; 1.2025x over previous
import jax
import jax.numpy as jnp
from jax.experimental import pallas as pl
from jax.experimental.pallas import tpu as pltpu

N_NODES = 16384
OBS_DIM = 4096
NUM_EXPERTS = 10
TOP_K = 3
ROW_BLOCK = 512
HALF = N_NODES // 2


def _hive_kernel(xa_ref, xb_ref, w_ref, b_ref, eps_ref,
                 w_out, l_out, v_out, i_out, acc_ref):
    step = pl.program_id(0)

    @pl.when(step == 0)
    def _init():
        acc_ref[...] = jnp.zeros_like(acc_ref)

    acc_ref[...] += (jnp.sum(xa_ref[...], axis=0, keepdims=True)
                     + jnp.sum(xb_ref[...], axis=0, keepdims=True))

    @pl.when(step == pl.num_programs(0) - 1)
    def _finish():
        gs = acc_ref[...] * (1.0 / N_NODES)
        both = jnp.dot(gs, w_ref[...],
                       preferred_element_type=jnp.float32) + b_ref[...]
        clean = both[:, :NUM_EXPERTS]
        raw = both[:, NUM_EXPERTS:]
        noise_std = jnp.logaddexp(raw, 0.0)
        logits = clean + eps_ref[...] * noise_std
        m = jnp.max(logits, axis=-1, keepdims=True)
        e = jnp.exp(logits - m)
        weights = e / jnp.sum(e, axis=-1, keepdims=True)
        l_out[...] = logits
        w_out[...] = weights

        idxs = jax.lax.broadcasted_iota(jnp.int32, (1, NUM_EXPERTS), 1)
        cur = weights
        vals, inds = [], []
        for _ in range(TOP_K):
            v = jnp.max(cur, axis=-1, keepdims=True)
            a = jnp.min(jnp.where(cur == v, idxs, NUM_EXPERTS),
                        axis=-1, keepdims=True)
            cur = jnp.where(idxs == a, -jnp.inf, cur)
            vals.append(v)
            inds.append(a)
        v_out[...] = jnp.concatenate(vals, axis=-1)
        i_out[...] = jnp.concatenate(inds, axis=-1)


@jax.jit
def _run(ip_x, w_both, b_both, eps):
    n_steps = HALF // ROW_BLOCK
    half_blocks = HALF // ROW_BLOCK
    grid = (n_steps,)
    full = lambda shape: pl.BlockSpec(shape, lambda i: (0,) * len(shape))
    out = pl.pallas_call(
        _hive_kernel,
        grid=grid,
        in_specs=[
            pl.BlockSpec((ROW_BLOCK, OBS_DIM), lambda i: (i, 0)),
            pl.BlockSpec((ROW_BLOCK, OBS_DIM),
                         lambda i: (half_blocks + i, 0)),
            full((OBS_DIM, 2 * NUM_EXPERTS)),
            full((1, 2 * NUM_EXPERTS)),
            full((1, NUM_EXPERTS)),
        ],
        out_specs=[
            full((1, NUM_EXPERTS)),
            full((1, NUM_EXPERTS)),
            full((1, TOP_K)),
            full((1, TOP_K)),
        ],
        out_shape=[
            jax.ShapeDtypeStruct((1, NUM_EXPERTS), jnp.float32),
            jax.ShapeDtypeStruct((1, NUM_EXPERTS), jnp.float32),
            jax.ShapeDtypeStruct((1, TOP_K), jnp.float32),
            jax.ShapeDtypeStruct((1, TOP_K), jnp.int32),
        ],
        scratch_shapes=[pltpu.VMEM((1, OBS_DIM), jnp.float32)],
        compiler_params=pltpu.CompilerParams(
            dimension_semantics=("arbitrary",),
        ),
    )(ip_x, ip_x, w_both, b_both, eps)
    weights, logits, top_k_vals, top_k_indices = out
    return weights, logits, top_k_vals, top_k_indices


def kernel(ip_x, w_gating, b_gating, w_noise, b_noise, top_k):
    eps = jax.random.normal(jax.random.key(42), (1, NUM_EXPERTS),
                            dtype=jnp.float32)
    w_both = jnp.concatenate([w_gating, w_noise], axis=1)
    b_both = jnp.concatenate([b_gating, b_noise]).reshape(1, 2 * NUM_EXPERTS)
    del top_k
    return _run(ip_x, w_both, b_both, eps)
